# baseline (speedup 1.0000x reference)
import functools

import jax
import jax.numpy as jnp
from jax import lax
from jax.experimental import pallas as pl
from jax.experimental.pallas import tpu as pltpu
from jax.experimental.pallas import tpu_sc as plsc

EMB = 64
K = 1024
L = 4
N = 32768
T = 1024
COMMIT = 0.25

_info = plsc.get_sparse_core_info()
_NW = _info.num_cores * _info.num_subcores
_BPW = N // _NW


def _argmin_math(r, cb):
    rn = jnp.sum(r * r, axis=1, keepdims=True)
    cn = jnp.sum(cb * cb, axis=1)[None, :]
    mm = jax.lax.dot_general(
        r, cb, (((1,), (1,)), ((), ())),
        preferred_element_type=jnp.float32)
    minv = rn + cn[:, :128] - 2.0 * mm[:, :128]
    mini = jnp.zeros((T, 128), dtype=jnp.int32)
    for s in range(1, K // 128):
        d_s = rn + cn[:, s * 128:(s + 1) * 128] - 2.0 * mm[:, s * 128:(s + 1) * 128]
        lt = d_s < minv
        mini = jnp.where(lt, s, mini)
        minv = jnp.where(lt, d_s, minv)
    lane128 = jax.lax.broadcasted_iota(jnp.int32, (T, 128), 1)
    kidx = mini * 128 + lane128
    m = jnp.min(minv, axis=1, keepdims=True)
    idx = jnp.min(jnp.where(minv == m, kidx, K), axis=1)
    return idx


def _argmin_first_block(r_ref, cb_ref, codes_ref):
    idx = _argmin_math(r_ref[...], cb_ref[...])
    codes_ref[...] = idx[None, :]


def _argmin_rest_block(r_ref, q_ref, cb_ref, rout_ref, codes_ref):
    r = r_ref[...] - q_ref[:, :EMB]
    rout_ref[...] = r
    idx = _argmin_math(r, cb_ref[...])
    codes_ref[...] = idx[None, :]


def _tc_argmin_first(r, cb):
    return pl.pallas_call(
        _argmin_first_block,
        grid=(N // T,),
        in_specs=[
            pl.BlockSpec((T, EMB), lambda i: (i, 0)),
            pl.BlockSpec((K, EMB), lambda i: (0, 0)),
        ],
        out_specs=pl.BlockSpec((1, T), lambda i: (0, i)),
        out_shape=jax.ShapeDtypeStruct((1, N), jnp.int32),
    )(r, cb)


def _tc_argmin_rest(r_prev, q_prev, cb):
    return pl.pallas_call(
        _argmin_rest_block,
        grid=(N // T,),
        in_specs=[
            pl.BlockSpec((T, EMB), lambda i: (i, 0)),
            pl.BlockSpec((T, 128), lambda i: (i, 0)),
            pl.BlockSpec((K, EMB), lambda i: (0, 0)),
        ],
        out_specs=[
            pl.BlockSpec((T, EMB), lambda i: (i, 0)),
            pl.BlockSpec((1, T), lambda i: (0, i)),
        ],
        out_shape=[
            jax.ShapeDtypeStruct((N, EMB), jnp.float32),
            jax.ShapeDtypeStruct((1, N), jnp.int32),
        ],
    )(r_prev, q_prev, cb)


_CH = _BPW // 2


def _sc_gather_body(table_hbm, idx_hbm, out_hbm, idx_v, rows_v, sem):
    wid = lax.axis_index("s") * _info.num_cores + lax.axis_index("c")
    base = wid * _BPW
    for h in range(_BPW // _CH):
        pltpu.sync_copy(idx_hbm.at[pl.ds(base + h * _CH, _CH)], idx_v)
        pltpu.async_copy(table_hbm.at[idx_v], rows_v, sem).wait()
        pltpu.sync_copy(rows_v, out_hbm.at[pl.ds(base + h * _CH, _CH)])


_sc_gather = functools.partial(
    pl.kernel,
    mesh=plsc.VectorSubcoreMesh(core_axis_name="c", subcore_axis_name="s"),
    out_type=jax.ShapeDtypeStruct((N, 128), jnp.float32),
    scratch_types=[
        pltpu.VMEM((_CH,), jnp.int32),
        pltpu.VMEM((_CH, 128), jnp.float32),
        pltpu.SemaphoreType.DMA,
    ],
)(_sc_gather_body)


def _combine_block(z_ref, q0_ref, q1_ref, q2_ref, q3_ref, qst_ref, loss_ref):
    step = pl.program_id(0)
    nsteps = pl.num_programs(0)
    z = z_ref[...]
    r = z
    qsum = jnp.zeros_like(z)
    part = jnp.zeros((1, 1), dtype=jnp.float32)
    for q_ref in (q0_ref, q1_ref, q2_ref, q3_ref):
        q = q_ref[:, :EMB]
        rq = r - q
        part = part + jnp.sum(rq * rq, axis=(0, 1), keepdims=True)
        qsum = qsum + q
        r = rq
    qst_ref[...] = z + (qsum - z)

    @pl.when(step == 0)
    def _init():
        loss_ref[...] = jnp.zeros_like(loss_ref)

    loss_ref[...] += jnp.broadcast_to(part, (8, 128))

    @pl.when(step == nsteps - 1)
    def _finish():
        loss_ref[...] = loss_ref[...] * (COMMIT / (N * EMB))


def _tc_combine(z, qs):
    return pl.pallas_call(
        _combine_block,
        grid=(N // T,),
        in_specs=([pl.BlockSpec((T, EMB), lambda i: (i, 0))]
                  + [pl.BlockSpec((T, 128), lambda i: (i, 0))] * 4),
        out_specs=[
            pl.BlockSpec((T, EMB), lambda i: (i, 0)),
            pl.BlockSpec((8, 128), lambda i: (0, 0)),
        ],
        out_shape=[
            jax.ShapeDtypeStruct((N, EMB), jnp.float32),
            jax.ShapeDtypeStruct((8, 128), jnp.float32),
        ],
    )(z, *qs)


@functools.partial(jax.jit, static_argnames=())
def kernel(inputs, codebooks):
    cbpad = jnp.pad(codebooks, ((0, 0), (0, 0), (0, 128 - EMB)))
    qs = []
    codes = []
    r = inputs
    q_prev = None
    for i in range(L):
        cb = codebooks[i]
        if i == 0:
            codes_i = _tc_argmin_first(r, cb)
        else:
            r, codes_i = _tc_argmin_rest(r, q_prev, cb)
        q_prev = _sc_gather(cbpad[i], codes_i.reshape(N))
        qs.append(q_prev)
        codes.append(codes_i)
    qst, loss = _tc_combine(inputs, qs)
    codes_tensor = jnp.concatenate(codes, axis=0).T
    return qst, loss[0, 0], codes_tensor


# speedup vs baseline: 1.6387x; 1.6387x over previous
import functools

import jax
import jax.numpy as jnp
from jax.experimental import pallas as pl
from jax.experimental.pallas import tpu as pltpu

EMB = 64
K = 1024
L = 4
N = 32768
T = 2048
COMMIT = 0.25


def _rvq_block(z_ref, cb_ref, cbp_ref, qst_ref, codes_ref, loss_ref):
    step = pl.program_id(0)
    nsteps = pl.num_programs(0)
    z = z_ref[...]
    r = z
    qsum = jnp.zeros_like(z)
    lane = jax.lax.broadcasted_iota(jnp.int32, (T, K), 1)
    lane128 = jax.lax.broadcasted_iota(jnp.int32, (T, 128), 1)
    part = jnp.zeros((1, 1), dtype=jnp.float32)
    codes_list = []
    for i in range(L):
        cb = cb_ref[i]
        rn = jnp.sum(r * r, axis=1, keepdims=True)
        cn = jnp.sum(cb * cb, axis=1)[None, :]
        mm = jax.lax.dot_general(
            r, cb, (((1,), (1,)), ((), ())),
            preferred_element_type=jnp.float32)
        rnb = jnp.broadcast_to(rn, (T, 128))
        minv = rnb + cn[:, :128] - 2.0 * mm[:, :128]
        mini = jnp.zeros((T, 128), dtype=jnp.int32)
        for s in range(1, K // 128):
            d_s = rnb + cn[:, s * 128:(s + 1) * 128] - 2.0 * mm[:, s * 128:(s + 1) * 128]
            lt = d_s < minv
            mini = jnp.where(lt, s, mini)
            minv = jnp.where(lt, d_s, minv)
        kidx = mini * 128 + lane128
        m = jnp.min(minv, axis=1, keepdims=True)
        idx = jnp.min(jnp.where(minv == m, kidx, K), axis=1)
        idxb = jnp.broadcast_to(idx[:, None], (T, 128))
        oh = jnp.concatenate(
            [(idxb == (lane128 + s * 128)) for s in range(K // 128)],
            axis=1).astype(jnp.bfloat16)
        qp = jax.lax.dot_general(
            oh, cbp_ref[i], (((1,), (0,)), ((), ())),
            preferred_element_type=jnp.float32)
        q = ((qp[:, :EMB] + qp[:, EMB:2 * EMB])
             + qp[:, 2 * EMB:3 * EMB]) + qp[:, 3 * EMB:]
        rq = r - q
        part = part + jnp.sum(rq * rq, axis=(0, 1), keepdims=True)
        qsum = qsum + q
        codes_list.append(idx)
        r = rq
    qst_ref[...] = z + (qsum - z)
    codes_ref[...] = jnp.stack(codes_list, axis=0)

    @pl.when(step == 0)
    def _init():
        loss_ref[...] = jnp.zeros_like(loss_ref)

    loss_ref[...] += jnp.broadcast_to(part, (8, 128))

    @pl.when(step == nsteps - 1)
    def _finish():
        loss_ref[...] = loss_ref[...] * (COMMIT / (N * EMB))


@functools.partial(jax.jit, static_argnames=())
def kernel(inputs, codebooks):
    nb = N // T
    b1 = codebooks.astype(jnp.bfloat16)
    r1 = codebooks - b1.astype(jnp.float32)
    b2 = r1.astype(jnp.bfloat16)
    r2 = r1 - b2.astype(jnp.float32)
    b3 = r2.astype(jnp.bfloat16)
    b4 = (r2 - b3.astype(jnp.float32)).astype(jnp.bfloat16)
    cbp = jnp.concatenate([b1, b2, b3, b4], axis=2)
    qst, codes_ln, loss = pl.pallas_call(
        _rvq_block,
        grid=(nb,),
        in_specs=[
            pl.BlockSpec((T, EMB), lambda i: (i, 0)),
            pl.BlockSpec((L, K, EMB), lambda i: (0, 0, 0)),
            pl.BlockSpec((L, K, 4 * EMB), lambda i: (0, 0, 0)),
        ],
        out_specs=[
            pl.BlockSpec((T, EMB), lambda i: (i, 0)),
            pl.BlockSpec((L, T), lambda i: (0, i)),
            pl.BlockSpec((8, 128), lambda i: (0, 0)),
        ],
        out_shape=[
            jax.ShapeDtypeStruct((N, EMB), jnp.float32),
            jax.ShapeDtypeStruct((L, N), jnp.int32),
            jax.ShapeDtypeStruct((8, 128), jnp.float32),
        ],
    )(inputs, codebooks, cbp)
    return qst, loss[0, 0], codes_ln.T
